# baseline (speedup 1.0000x reference)
import functools

import jax
import jax.numpy as jnp
from jax import lax
from jax.experimental import pallas as pl
from jax.experimental.pallas import tpu as pltpu
from jax.experimental.pallas import tpu_sc as plsc

_NC = 2
_NS = 16
_NW = _NC * _NS

_IDX_W = 8


@functools.partial(jax.jit, static_argnames=("n_ctx", "suf_len", "d"))
def _sc_gather(prefix, ctx, suffix, idx_flat, *, n_ctx, suf_len, d):
    b = idx_flat.shape[0] // _IDX_W
    nb = b // _NW
    seq = 1 + n_ctx + suf_len
    mesh = plsc.VectorSubcoreMesh(
        core_axis_name="c", subcore_axis_name="s",
        num_cores=_NC, num_subcores=_NS,
    )

    @functools.partial(
        pl.kernel,
        out_type=jax.ShapeDtypeStruct((b, seq, d), jnp.float32),
        mesh=mesh,
        compiler_params=pltpu.CompilerParams(use_tc_tiling_on_sc=False),
        scratch_types=[
            pltpu.VMEM((nb * _IDX_W,), jnp.int32),
            [pltpu.VMEM((1, 1, d), jnp.float32),
             pltpu.VMEM((1, n_ctx, d), jnp.float32),
             pltpu.VMEM((1, suf_len, d), jnp.float32)],
            [pltpu.VMEM((1, 1, d), jnp.float32),
             pltpu.VMEM((1, n_ctx, d), jnp.float32),
             pltpu.VMEM((1, suf_len, d), jnp.float32)],
            pltpu.SemaphoreType.DMA,
            pltpu.SemaphoreType.DMA,
            pltpu.SemaphoreType.DMA,
            pltpu.SemaphoreType.DMA,
        ],
    )
    def k(pre_hbm, ctx_hbm, suf_hbm, idx_hbm, out_hbm,
          idx_v, bufs0, bufs1, g0, g1, s0, s1):
        wid = lax.axis_index("s") * _NC + lax.axis_index("c")
        base = wid * nb
        pltpu.sync_copy(
            idx_hbm.at[pl.ds(pl.multiple_of(base * _IDX_W, 8), nb * _IDX_W)],
            idx_v)

        bufs = (bufs0, bufs1)
        gsems = (g0, g1)
        ssems = (s0, s1)
        tables = (pre_hbm, ctx_hbm, suf_hbm)

        def gathers(i, p):
            off = pl.multiple_of(i * _IDX_W, 8)
            lab = idx_v.at[pl.ds(off, 1)]
            return [pltpu.make_async_copy(t.at[lab], dst, gsems[p])
                    for t, dst in zip(tables, bufs[p])]

        def stores(i, p):
            row = out_hbm.at[pl.ds(base + i, 1)]
            dsts = (row.at[:, pl.ds(0, 1)],
                    row.at[:, pl.ds(1, n_ctx)],
                    row.at[:, pl.ds(1 + n_ctx, suf_len)])
            return [pltpu.make_async_copy(src, dst, ssems[p])
                    for src, dst in zip(bufs[p], dsts)]

        def fire(cs):
            for c in cs:
                c.start()

        def drain(cs):
            for c in cs:
                c.wait()

        fire(gathers(0, 0))

        def body(j, _):
            i = j * 2
            @pl.when(j > 0)
            def _():
                drain(stores(i - 1, 1))
            fire(gathers(i + 1, 1))
            drain(gathers(i, 0))
            fire(stores(i, 0))
            @pl.when(j < nb // 2 - 1)
            def _():
                drain(stores(i, 0))
                fire(gathers(i + 2, 0))
            drain(gathers(i + 1, 1))
            fire(stores(i + 1, 1))
            return 0

        lax.fori_loop(0, nb // 2, body, 0)
        drain(stores(nb - 2, 0))
        drain(stores(nb - 1, 1))

    return k(prefix, ctx, suffix, idx_flat)


def kernel(labels, ctx, token_prefix, token_suffix):
    n_cls, n_ctx, d = ctx.shape
    suf_len = token_suffix.shape[1]
    lab = labels.astype(jnp.int32)
    idx = jnp.repeat(lab, _IDX_W)
    return _sc_gather(token_prefix, ctx, token_suffix, idx,
                      n_ctx=n_ctx, suf_len=suf_len, d=d)


# speedup vs baseline: 1.7526x; 1.7526x over previous
import functools

import jax
import jax.numpy as jnp
from jax import lax
from jax.experimental import pallas as pl
from jax.experimental.pallas import tpu as pltpu
from jax.experimental.pallas import tpu_sc as plsc

_NC = 2
_NS = 16
_NW = _NC * _NS

_IDX_W = 16

_LANES = 16
_QUART = 128


@functools.partial(jax.jit, static_argnames=("n_ctx", "suf_len", "d"))
def _sc_gather(prefix, ctx, suffix, idx_flat, *, n_ctx, suf_len, d):
    b = idx_flat.shape[0] // _IDX_W
    nb = b // _NW
    seq = 1 + n_ctx + suf_len
    n_q = d // _QUART
    mesh = plsc.VectorSubcoreMesh(
        core_axis_name="c", subcore_axis_name="s",
        num_cores=_NC, num_subcores=_NS,
    )

    @functools.partial(
        pl.kernel,
        out_type=jax.ShapeDtypeStruct((b, seq, d), jnp.float32),
        mesh=mesh,
        compiler_params=pltpu.CompilerParams(needs_layout_passes=False),
        scratch_types=[
            pltpu.VMEM((nb * _IDX_W,), jnp.int32),
            [pltpu.VMEM((1, 1, d), jnp.float32),
             pltpu.VMEM((1, n_ctx, d), jnp.float32),
             pltpu.VMEM((1, suf_len, d), jnp.float32)],
            [pltpu.VMEM((1, 1, d), jnp.float32),
             pltpu.VMEM((1, n_ctx, d), jnp.float32),
             pltpu.VMEM((1, suf_len, d), jnp.float32)],
            pltpu.VMEM((1, seq, _QUART), jnp.float32),
            pltpu.VMEM((1, seq, _QUART), jnp.float32),
            pltpu.SemaphoreType.DMA,
            pltpu.SemaphoreType.DMA,
            pltpu.SemaphoreType.DMA,
            pltpu.SemaphoreType.DMA,
        ],
    )
    def k(pre_hbm, ctx_hbm, suf_hbm, idx_hbm, out_hbm,
          idx_v, bufs0, bufs1, row0, row1, g0, g1, s0, s1):
        wid = lax.axis_index("s") * _NC + lax.axis_index("c")
        base = wid * nb
        pltpu.sync_copy(
            idx_hbm.at[pl.ds(pl.multiple_of(base * _IDX_W, 8), nb * _IDX_W)],
            idx_v)

        bufs = (bufs0, bufs1)
        rows = (row0, row1)
        gsems = (g0, g1)
        ssems = (s0, s1)
        tables = (pre_hbm, ctx_hbm, suf_hbm)

        def gathers(e, p):
            off = pl.multiple_of(e * _IDX_W, 8)
            lab = lax.reduce_max(idx_v[pl.ds(off, _LANES)], (0,))
            return [pltpu.make_async_copy(t.at[pl.ds(lab, 1)], dst, gsems[p])
                    for t, dst in zip(tables, bufs[p])]

        def store(e, q, r):
            return [pltpu.make_async_copy(
                rows[r],
                out_hbm.at[pl.ds(base + e, 1), :, pl.ds(q * _QUART, _QUART)],
                ssems[r])]

        def fire(cs):
            for c in cs:
                c.start()

        def drain(cs):
            for c in cs:
                c.wait()

        def assemble(p, q, r):
            pre_b, ctx_b, suf_b = bufs[p]
            row_b = rows[r]
            qb = q * _QUART
            for c in range(_QUART // _LANES):
                cs = pl.ds(qb + c * _LANES, _LANES)
                ds = pl.ds(c * _LANES, _LANES)
                row_b[0, 0, ds] = pre_b[0, 0, cs]

            def ctx_body(t, _):
                for c in range(_QUART // _LANES):
                    cs = pl.ds(qb + c * _LANES, _LANES)
                    ds = pl.ds(c * _LANES, _LANES)
                    row_b[0, t + 1, ds] = ctx_b[0, t, cs]
                return 0

            lax.fori_loop(0, n_ctx, ctx_body, 0)

            def suf_body(t, _):
                for c in range(_QUART // _LANES):
                    cs = pl.ds(qb + c * _LANES, _LANES)
                    ds = pl.ds(c * _LANES, _LANES)
                    row_b[0, t + (1 + n_ctx), ds] = suf_b[0, t, cs]
                return 0

            lax.fori_loop(0, suf_len, suf_body, 0)

        fire(gathers(0, 0))

        def element(j, e, p):
            @pl.when(e + 1 < nb)
            def _():
                fire(gathers(e + 1, 1 - p))
            drain(gathers(e, p))
            for q in range(n_q):
                r = q % 2
                if p > 0 or q >= 2:
                    drain(store(e, q, r))
                else:
                    @pl.when(j > 0)
                    def _(q=q, r=r):
                        drain(store(e, q, r))
                assemble(p, q, r)
                fire(store(e, q, r))

        def body(j, _):
            e = j * 2
            element(j, e, 0)
            element(j, e + 1, 1)
            return 0

        lax.fori_loop(0, nb // 2, body, 0)
        drain(store(nb - 1, n_q - 2, 0))
        drain(store(nb - 1, n_q - 1, 1))

    return k(prefix, ctx, suffix, idx_flat)


def kernel(labels, ctx, token_prefix, token_suffix):
    n_cls, n_ctx, d = ctx.shape
    suf_len = token_suffix.shape[1]
    lab = labels.astype(jnp.int32)
    idx = jnp.repeat(lab, _IDX_W)
    return _sc_gather(token_prefix, ctx, token_suffix, idx,
                      n_ctx=n_ctx, suf_len=suf_len, d=d)


# speedup vs baseline: 2.5879x; 1.4767x over previous
import functools

import jax
import jax.numpy as jnp
from jax import lax
from jax.experimental import pallas as pl
from jax.experimental.pallas import tpu as pltpu
from jax.experimental.pallas import tpu_sc as plsc

_NC = 2
_NS = 16
_NW = _NC * _NS

_IDX_W = 16

_LANES = 16
_QUART = 128


@functools.partial(jax.jit, static_argnames=("n_ctx", "suf_len", "d"))
def _sc_gather(prefix, ctx, suffix, idx_flat, *, n_ctx, suf_len, d):
    b = idx_flat.shape[0] // _IDX_W
    nb = b // _NW
    seq = 1 + n_ctx + suf_len
    n_q = d // _QUART
    mesh = plsc.VectorSubcoreMesh(
        core_axis_name="c", subcore_axis_name="s",
        num_cores=_NC, num_subcores=_NS,
    )

    @functools.partial(
        pl.kernel,
        out_type=jax.ShapeDtypeStruct((b, seq, d), jnp.float32),
        mesh=mesh,
        compiler_params=pltpu.CompilerParams(needs_layout_passes=False),
        scratch_types=[
            pltpu.VMEM((nb * _IDX_W,), jnp.int32),
            [pltpu.VMEM((1, 1, d), jnp.float32),
             pltpu.VMEM((1, n_ctx, d), jnp.float32),
             pltpu.VMEM((1, suf_len, d), jnp.float32)],
            [pltpu.VMEM((1, 1, d), jnp.float32),
             pltpu.VMEM((1, n_ctx, d), jnp.float32),
             pltpu.VMEM((1, suf_len, d), jnp.float32)],
            pltpu.VMEM((1, seq, _QUART), jnp.float32),
            pltpu.VMEM((1, seq, _QUART), jnp.float32),
            pltpu.SemaphoreType.DMA,
            pltpu.SemaphoreType.DMA,
            pltpu.SemaphoreType.DMA,
            pltpu.SemaphoreType.DMA,
        ],
    )
    def k(pre_hbm, ctx_hbm, suf_hbm, idx_hbm, out_hbm,
          idx_v, bufs0, bufs1, row0, row1, g0, g1, s0, s1):
        wid = lax.axis_index("s") * _NC + lax.axis_index("c")
        base = wid * nb
        pltpu.sync_copy(
            idx_hbm.at[pl.ds(pl.multiple_of(base * _IDX_W, 8), nb * _IDX_W)],
            idx_v)

        bufs = (bufs0, bufs1)
        rows = (row0, row1)
        gsems = (g0, g1)
        ssems = (s0, s1)
        tables = (pre_hbm, ctx_hbm, suf_hbm)

        def gathers(e, p):
            off = pl.multiple_of(e * _IDX_W, 8)
            lab = lax.reduce_max(idx_v[pl.ds(off, _LANES)], (0,))
            return [pltpu.make_async_copy(t.at[pl.ds(lab, 1)], dst, gsems[p])
                    for t, dst in zip(tables, bufs[p])]

        def store(e, q, r):
            return [pltpu.make_async_copy(
                rows[r],
                out_hbm.at[pl.ds(base + e, 1), :, pl.ds(q * _QUART, _QUART)],
                ssems[r])]

        def fire(cs):
            for c in cs:
                c.start()

        def drain(cs):
            for c in cs:
                c.wait()

        def assemble(p, q, r):
            pre_b, ctx_b, suf_b = bufs[p]
            row_b = rows[r]
            qb = q * _QUART

            def chunk_body(c, _):
                co = c * _LANES
                src = pl.ds(qb + co, _LANES)
                dst = pl.ds(co, _LANES)
                row_b[0, 0, dst] = pre_b[0, 0, src]
                for t in range(n_ctx):
                    row_b[0, t + 1, dst] = ctx_b[0, t, src]
                for t in range(suf_len):
                    row_b[0, t + (1 + n_ctx), dst] = suf_b[0, t, src]
                return 0

            lax.fori_loop(0, _QUART // _LANES, chunk_body, 0)

        fire(gathers(0, 0))

        def element(j, e, p):
            @pl.when(e + 1 < nb)
            def _():
                fire(gathers(e + 1, 1 - p))
            drain(gathers(e, p))
            for q in range(n_q):
                r = q % 2
                if p > 0 or q >= 2:
                    drain(store(e, q, r))
                else:
                    @pl.when(j > 0)
                    def _(q=q, r=r):
                        drain(store(e, q, r))
                assemble(p, q, r)
                fire(store(e, q, r))

        def body(j, _):
            e = j * 2
            element(j, e, 0)
            element(j, e + 1, 1)
            return 0

        lax.fori_loop(0, nb // 2, body, 0)
        drain(store(nb - 1, n_q - 2, 0))
        drain(store(nb - 1, n_q - 1, 1))

    return k(prefix, ctx, suffix, idx_flat)


def kernel(labels, ctx, token_prefix, token_suffix):
    n_cls, n_ctx, d = ctx.shape
    suf_len = token_suffix.shape[1]
    lab = labels.astype(jnp.int32)
    idx = jnp.repeat(lab, _IDX_W)
    return _sc_gather(token_prefix, ctx, token_suffix, idx,
                      n_ctx=n_ctx, suf_len=suf_len, d=d)


# speedup vs baseline: 2.5900x; 1.0008x over previous
import functools

import jax
import jax.numpy as jnp
from jax import lax
from jax.experimental import pallas as pl
from jax.experimental.pallas import tpu as pltpu
from jax.experimental.pallas import tpu_sc as plsc

_NC = 2
_NS = 16
_NW = _NC * _NS

_IDX_W = 16

_LANES = 16
_QUART = 128


@functools.partial(jax.jit, static_argnames=("n_ctx", "suf_len", "d"))
def _sc_gather(prefix, ctx, suffix, idx_flat, *, n_ctx, suf_len, d):
    b = idx_flat.shape[0] // _IDX_W
    nb = b // _NW
    seq = 1 + n_ctx + suf_len
    n_q = d // _QUART
    mesh = plsc.VectorSubcoreMesh(
        core_axis_name="c", subcore_axis_name="s",
        num_cores=_NC, num_subcores=_NS,
    )

    @functools.partial(
        pl.kernel,
        out_type=jax.ShapeDtypeStruct((b, seq, d), jnp.float32),
        mesh=mesh,
        compiler_params=pltpu.CompilerParams(needs_layout_passes=False),
        scratch_types=[
            pltpu.VMEM((nb * _IDX_W,), jnp.int32),
            [pltpu.VMEM((1, 1, d), jnp.float32),
             pltpu.VMEM((1, n_ctx, d), jnp.float32),
             pltpu.VMEM((1, suf_len, d), jnp.float32)],
            [pltpu.VMEM((1, 1, d), jnp.float32),
             pltpu.VMEM((1, n_ctx, d), jnp.float32),
             pltpu.VMEM((1, suf_len, d), jnp.float32)],
            pltpu.VMEM((1, seq, _QUART), jnp.float32),
            pltpu.VMEM((1, seq, _QUART), jnp.float32),
            pltpu.SemaphoreType.DMA,
            pltpu.SemaphoreType.DMA,
            pltpu.SemaphoreType.DMA,
            pltpu.SemaphoreType.DMA,
        ],
    )
    def k(pre_hbm, ctx_hbm, suf_hbm, idx_hbm, out_hbm,
          idx_v, bufs0, bufs1, row0, row1, g0, g1, s0, s1):
        wid = lax.axis_index("s") * _NC + lax.axis_index("c")
        base = wid * nb
        pltpu.sync_copy(
            idx_hbm.at[pl.ds(pl.multiple_of(base * _IDX_W, 8), nb * _IDX_W)],
            idx_v)

        bufs = (bufs0, bufs1)
        rows = (row0, row1)
        gsems = (g0, g1)
        ssems = (s0, s1)
        tables = (pre_hbm, ctx_hbm, suf_hbm)

        def gathers(e, p):
            off = pl.multiple_of(e * _IDX_W, 8)
            lab = lax.reduce_max(idx_v[pl.ds(off, _LANES)], (0,))
            return [pltpu.make_async_copy(t.at[pl.ds(lab, 1)], dst, gsems[p])
                    for t, dst in zip(tables, bufs[p])]

        def store(e, q, r):
            return [pltpu.make_async_copy(
                rows[r],
                out_hbm.at[pl.ds(base + e, 1), :, pl.ds(q * _QUART, _QUART)],
                ssems[r])]

        def fire(cs):
            for c in cs:
                c.start()

        def drain(cs):
            for c in cs:
                c.wait()

        def assemble(p, q, r):
            pre_b, ctx_b, suf_b = bufs[p]
            row_b = rows[r]
            qb = q * _QUART

            moves = ([(0, pre_b, 0)]
                     + [(t + 1, ctx_b, t) for t in range(n_ctx)]
                     + [(t + 1 + n_ctx, suf_b, t) for t in range(suf_len)])

            def chunk_body(c, _):
                co = c * _LANES
                src = pl.ds(qb + co, _LANES)
                dst = pl.ds(co, _LANES)
                for g in range(0, len(moves), 16):
                    batch = moves[g:g + 16]
                    vals = [sb[0, sr, src] for _, sb, sr in batch]
                    for (dr, _, _), v in zip(batch, vals):
                        row_b[0, dr, dst] = v
                return 0

            lax.fori_loop(0, _QUART // _LANES, chunk_body, 0)

        fire(gathers(0, 0))

        def element(j, e, p):
            @pl.when(e + 1 < nb)
            def _():
                fire(gathers(e + 1, 1 - p))
            drain(gathers(e, p))
            for q in range(n_q):
                r = q % 2
                if p > 0 or q >= 2:
                    drain(store(e, q, r))
                else:
                    @pl.when(j > 0)
                    def _(q=q, r=r):
                        drain(store(e, q, r))
                assemble(p, q, r)
                fire(store(e, q, r))

        def body(j, _):
            e = j * 2
            element(j, e, 0)
            element(j, e + 1, 1)
            return 0

        lax.fori_loop(0, nb // 2, body, 0)
        drain(store(nb - 1, n_q - 2, 0))
        drain(store(nb - 1, n_q - 1, 1))

    return k(prefix, ctx, suffix, idx_flat)


def kernel(labels, ctx, token_prefix, token_suffix):
    n_cls, n_ctx, d = ctx.shape
    suf_len = token_suffix.shape[1]
    lab = labels.astype(jnp.int32)
    idx = jnp.repeat(lab, _IDX_W)
    return _sc_gather(token_prefix, ctx, token_suffix, idx,
                      n_ctx=n_ctx, suf_len=suf_len, d=d)


# speedup vs baseline: 6.4177x; 2.4779x over previous
import functools

import jax
import jax.numpy as jnp
from jax import lax
from jax.experimental import pallas as pl
from jax.experimental.pallas import tpu as pltpu
from jax.experimental.pallas import tpu_sc as plsc

_NC = 2
_NS = 16
_NW = _NC * _NS

_LANES = 16
_NBUF = 3


@functools.partial(jax.jit, static_argnames=("n_ctx", "suf_len", "d"))
def _sc_gather(prefix2d, ctx2d, suffix_t, labels, *, n_ctx, suf_len, d):
    b = labels.shape[0]
    k = b // _NW
    seq = 1 + n_ctx + suf_len
    mesh = plsc.VectorSubcoreMesh(
        core_axis_name="c", subcore_axis_name="s",
        num_cores=_NC, num_subcores=_NS,
    )

    @functools.partial(
        pl.kernel,
        out_type=jax.ShapeDtypeStruct((seq, b, d), jnp.float32),
        mesh=mesh,
        compiler_params=pltpu.CompilerParams(needs_layout_passes=False),
        scratch_types=(
            [pltpu.VMEM((k,), jnp.int32)]
            + [pltpu.VMEM((k, d), jnp.float32) for _ in range(_NBUF)]
            + [pltpu.SemaphoreType.DMA for _ in range(2 * _NBUF)]
        ),
    )
    def kfn(pre_hbm, ctx_hbm, suf_hbm, lab_hbm, out_hbm, idx_v, *rest):
        bufs = rest[:_NBUF]
        gsems = rest[_NBUF:2 * _NBUF]
        ssems = rest[2 * _NBUF:]
        wid = lax.axis_index("s") * _NC + lax.axis_index("c")
        b0 = wid * k
        pltpu.sync_copy(lab_hbm.at[pl.ds(pl.multiple_of(b0, 8), k)], idx_v)

        def gathers(s, p):
            if s == 0:
                return [pltpu.make_async_copy(
                    pre_hbm.at[idx_v], bufs[p], gsems[p])]
            if s <= n_ctx:
                r = s - 1
                cs = []
                for g in range(k // _LANES):
                    vec = idx_v[pl.ds(g * _LANES, _LANES)] * n_ctx + r
                    cs.append(pltpu.make_async_copy(
                        ctx_hbm.at[vec],
                        bufs[p].at[pl.ds(g * _LANES, _LANES)], gsems[p]))
                return cs
            return [pltpu.make_async_copy(
                suf_hbm.at[s - (1 + n_ctx)].at[idx_v], bufs[p], gsems[p])]

        def store(s, p):
            return [pltpu.make_async_copy(
                bufs[p],
                out_hbm.at[s, pl.ds(pl.multiple_of(b0, 8), k), :],
                ssems[p])]

        def fire(cs):
            for c in cs:
                c.start()

        def drain(cs):
            for c in cs:
                c.wait()

        fire(gathers(0, 0))
        for s in range(seq):
            if s + 1 < seq:
                if s >= _NBUF - 1:
                    drain(store(s - (_NBUF - 1), (s + 1) % _NBUF))
                fire(gathers(s + 1, (s + 1) % _NBUF))
            drain(gathers(s, s % _NBUF))
            fire(store(s, s % _NBUF))
        for s in range(seq - _NBUF, seq):
            drain(store(s, s % _NBUF))

    return kfn(prefix2d, ctx2d, suffix_t, labels)


def kernel(labels, ctx, token_prefix, token_suffix):
    n_cls, n_ctx, d = ctx.shape
    suf_len = token_suffix.shape[1]
    b = labels.shape[0]
    lab = labels.astype(jnp.int32)
    out_t = _sc_gather(
        token_prefix.reshape(n_cls, d),
        ctx.reshape(n_cls * n_ctx, d),
        jnp.transpose(token_suffix, (1, 0, 2)),
        lab,
        n_ctx=n_ctx, suf_len=suf_len, d=d)
    return jnp.transpose(out_t, (1, 0, 2))
